# Initial kernel scaffold; baseline (speedup 1.0000x reference)
import jax
import jax.numpy as jnp
from jax.experimental import pallas as pl


def kernel(tampering_codes, table, W1, b1, W2, b2):
    raise NotImplementedError("write your pallas kernel here")


# speedup vs baseline: 1.1462x; 1.1462x over previous
import functools

import jax
import jax.numpy as jnp
from jax import lax
from jax.experimental import pallas as pl
from jax.experimental.pallas import tpu as pltpu
from jax.experimental.pallas import tpu_sc as plsc

D = 64
NUM_CORES = 2
NUM_SUBCORES = 16
NUM_WORKERS = NUM_CORES * NUM_SUBCORES

GROUP_ROWS = 128
GROUPS = 8
CHUNK = GROUP_ROWS * GROUPS


def _gather_body(tab_hbm, idx_hbm, out_hbm, idx_v, rows_v, sem, *, rows_per_w):
    wid = lax.axis_index("s") * NUM_CORES + lax.axis_index("c")
    base = wid * rows_per_w
    num_chunks = rows_per_w // CHUNK

    def chunk_body(c, carry):
        cbase = pl.multiple_of(base + c * CHUNK, CHUNK)
        row0 = pl.multiple_of(cbase // GROUP_ROWS, GROUPS)
        pltpu.sync_copy(idx_hbm.at[pl.ds(row0, GROUPS)], idx_v)
        copies = [
            pltpu.async_copy(
                tab_hbm.at[idx_v.at[g]],
                rows_v.at[pl.ds(g * GROUP_ROWS, GROUP_ROWS)],
                sem,
            )
            for g in range(GROUPS)
        ]
        for cp in copies:
            cp.wait()
        pltpu.sync_copy(rows_v, out_hbm.at[pl.ds(cbase, CHUNK)])
        return carry

    lax.fori_loop(0, num_chunks, chunk_body, 0)


def _make_gather(n_rows):
    rows_per_w = n_rows // NUM_WORKERS
    body = functools.partial(_gather_body, rows_per_w=rows_per_w)
    return pl.kernel(
        body,
        mesh=plsc.VectorSubcoreMesh(core_axis_name="c", subcore_axis_name="s"),
        compiler_params=pltpu.CompilerParams(use_tc_tiling_on_sc=False),
        out_type=jax.ShapeDtypeStruct((n_rows, D), jnp.float32),
        scratch_types=[
            pltpu.VMEM((GROUPS, GROUP_ROWS), jnp.int32),
            pltpu.VMEM((CHUNK, D), jnp.float32),
            pltpu.SemaphoreType.DMA,
        ],
    )


def _mlp_body(x_ref, w1_ref, b1_ref, w2_ref, b2_ref, o_ref):
    x = x_ref[...]
    h = lax.dot_general(
        x, w1_ref[...], (((1,), (1,)), ((), ())),
        preferred_element_type=jnp.float32,
    )
    h = jnp.maximum(h + b1_ref[...], 0.0)
    o = lax.dot_general(
        h, w2_ref[...], (((1,), (1,)), ((), ())),
        preferred_element_type=jnp.float32,
    )
    o_ref[...] = o + b2_ref[...]


def _mlp(x, W1, b1, W2, b2, block_rows):
    n = x.shape[0]
    grid = n // block_rows
    return pl.pallas_call(
        _mlp_body,
        grid=(grid,),
        in_specs=[
            pl.BlockSpec((block_rows, D), lambda i: (i, 0)),
            pl.BlockSpec((D, D), lambda i: (0, 0)),
            pl.BlockSpec((1, D), lambda i: (0, 0)),
            pl.BlockSpec((D, D), lambda i: (0, 0)),
            pl.BlockSpec((1, D), lambda i: (0, 0)),
        ],
        out_specs=pl.BlockSpec((block_rows, D), lambda i: (i, 0)),
        out_shape=jax.ShapeDtypeStruct((n, D), jnp.float32),
    )(x, W1, b1, W2, b2)


def kernel(tampering_codes, table, W1, b1, W2, b2):
    B, L = tampering_codes.shape
    n = B * L
    idx = tampering_codes.reshape(n // GROUP_ROWS, GROUP_ROWS).astype(jnp.int32)
    emb = _make_gather(n)(table, idx)
    out = _mlp(emb, W1, b1.reshape(1, D), W2, b2.reshape(1, D), block_rows=4096)
    return out.reshape(B, L, D)
